# Initial kernel scaffold; baseline (speedup 1.0000x reference)
import jax
import jax.numpy as jnp
from jax.experimental import pallas as pl


def kernel(z, code, edge_index, W1, b1, W2, b2):
    raise NotImplementedError("write your pallas kernel here")


# speedup vs baseline: 9.4608x; 9.4608x over previous
import functools

import jax
import jax.numpy as jnp
from jax import lax
from jax.experimental import pallas as pl
from jax.experimental.pallas import tpu as pltpu
from jax.experimental.pallas import tpu_sc as plsc

E = 32
N = 8192
ADJ = 4096
BETA = 0.25
NC, NS = 2, 16
NW = NC * NS
EDGE_COLS = 128

def _mm1_body(x_ref, w_ref, o_ref):
    o_ref[...] = jnp.dot(x_ref[...], w_ref[...],
                         preferred_element_type=jnp.float32)


def _mm1(code, W1):
    M, K = code.shape
    Nout = W1.shape[1]
    blk = 1024
    return pl.pallas_call(
        _mm1_body,
        grid=(M // blk,),
        in_specs=[pl.BlockSpec((blk, K), lambda i: (i, 0)),
                  pl.BlockSpec((K, Nout), lambda i: (0, 0))],
        out_specs=pl.BlockSpec((blk, Nout), lambda i: (i, 0)),
        out_shape=jax.ShapeDtypeStruct((M, Nout), jnp.float32),
    )(code, W1)


def _sc_degree(dst2d):
    rows_pt = dst2d.shape[0] // NW
    mesh = plsc.VectorSubcoreMesh(core_axis_name="c", subcore_axis_name="s")

    @functools.partial(
        pl.kernel, mesh=mesh,
        out_type=jax.ShapeDtypeStruct((NW, N), jnp.float32),
        scratch_types=[pltpu.VMEM((rows_pt, EDGE_COLS), jnp.int32),
                       pltpu.VMEM((N,), jnp.float32)],
        compiler_params=pltpu.CompilerParams(use_tc_tiling_on_sc=False,
                                             needs_layout_passes=False),
    )
    def k(dst_hbm, out_hbm, dstv, hist):
        c = lax.axis_index("c")
        s = lax.axis_index("s")
        wid = c * NS + s
        z16 = jnp.zeros((16,), jnp.float32)

        def zero_body(i, _):
            hist[pl.ds(i * 16, 16)] = z16
            return 0
        lax.fori_loop(0, N // 16, zero_body, 0)

        pltpu.sync_copy(dst_hbm.at[pl.ds(wid * rows_pt, rows_pt)], dstv)
        ones = jnp.ones((16,), jnp.float32)

        def body(r, _):
            for g in range(EDGE_COLS // 16):
                idx = dstv[r, pl.ds(g * 16, 16)]
                plsc.addupdate_scatter(hist, [idx], ones)
            return 0
        lax.fori_loop(0, rows_pt, body, 0)

        pltpu.sync_copy(hist, out_hbm.at[wid])

    return k(dst2d)


def _sc_scatter(hs, src2d, dst2d):
    rows_pt = src2d.shape[0] // NW
    rows_per_sub = N // NS
    mesh = plsc.VectorSubcoreMesh(core_axis_name="c", subcore_axis_name="s")

    @functools.partial(
        pl.kernel, mesh=mesh,
        out_type=jax.ShapeDtypeStruct((NC, N, E), jnp.float32),
        scratch_types=[
            pltpu.VMEM((rows_pt, EDGE_COLS), jnp.int32),
            pltpu.VMEM((rows_pt, EDGE_COLS), jnp.int32),
            pltpu.VMEM((EDGE_COLS, E), jnp.float32),
            pltpu.VMEM((EDGE_COLS, E), jnp.float32),
            pltpu.VMEM_SHARED((N, E), jnp.float32),
            pltpu.SemaphoreType.DMA,
        ],
        compiler_params=pltpu.CompilerParams(use_tc_tiling_on_sc=False),
    )
    def k(hs_hbm, src_hbm, dst_hbm, out_hbm, srcv, dstv, rows, zb, acc, sem):
        c = lax.axis_index("c")
        s = lax.axis_index("s")
        wid = c * NS + s
        z16 = jnp.zeros((16,), jnp.float32)

        def zb_body(i, _):
            zb[i, pl.ds(0, 16)] = z16
            zb[i, pl.ds(16, 16)] = z16
            return 0
        lax.fori_loop(0, EDGE_COLS, zb_body, 0)
        for t in range(rows_per_sub // EDGE_COLS):
            pltpu.sync_copy(zb, acc.at[pl.ds(s * rows_per_sub + t * EDGE_COLS,
                                             EDGE_COLS)])
        pltpu.sync_copy(src_hbm.at[pl.ds(wid * rows_pt, rows_pt)], srcv)
        pltpu.sync_copy(dst_hbm.at[pl.ds(wid * rows_pt, rows_pt)], dstv)
        plsc.subcore_barrier()

        def body(j, _):
            pltpu.async_copy(hs_hbm.at[srcv.at[j]], rows, sem).wait()
            pltpu.sync_copy(rows, acc.at[dstv.at[j]], add=True)
            return 0
        lax.fori_loop(0, rows_pt, body, 0)
        plsc.subcore_barrier()

        pltpu.sync_copy(acc.at[pl.ds(s * rows_per_sub, rows_per_sub)],
                        out_hbm.at[c, pl.ds(s * rows_per_sub, rows_per_sub)])

    return k(hs, src2d, dst2d)


def _prep_body(degp_ref, mm1_ref, dinv_ref, hs1_ref):
    deg = jnp.sum(degp_ref[...], axis=0) + 1.0
    dinv = lax.rsqrt(deg)
    dinv_ref[...] = dinv[:, None]
    hs1_ref[...] = mm1_ref[...] * dinv[:, None]


def _prep(degp, mm1):
    return pl.pallas_call(
        _prep_body,
        out_shape=[jax.ShapeDtypeStruct((N, 1), jnp.float32),
                   jax.ShapeDtypeStruct((N, E), jnp.float32)],
    )(degp, mm1)


def _layer2_body(accp_ref, hs1_ref, dinv_ref, b1_ref, w2_ref, hs2_ref):
    dinv = dinv_ref[...]
    h2 = dinv * (accp_ref[0] + accp_ref[1] + hs1_ref[...]) + b1_ref[...]
    h2 = jnp.maximum(h2, 0.0)
    hs2_ref[...] = jnp.dot(h2, w2_ref[...],
                           preferred_element_type=jnp.float32) * dinv


def _layer2(accp1, hs1, dinv, b1_2d, W2):
    return pl.pallas_call(
        _layer2_body,
        out_shape=jax.ShapeDtypeStruct((N, E), jnp.float32),
    )(accp1, hs1, dinv, b1_2d, W2)


def _codebooks_body(accp_ref, hs2_ref, dinv_ref, b2_ref,
                    ew_ref, ew2_ref, sq_ref, sq2_ref):
    total = (dinv_ref[...] * (accp_ref[0] + accp_ref[1] + hs2_ref[...])
             + b2_ref[...])
    ew = total[:ADJ]
    ew2 = total[ADJ:]
    ew_ref[...] = ew
    ew2_ref[...] = ew2
    sq_ref[...] = jnp.sum(ew ** 2, axis=1)[None, :]
    sq2_ref[...] = jnp.sum(ew2 ** 2, axis=1)[None, :]


def _codebooks(accp2, hs2, dinv, b2_2d):
    return pl.pallas_call(
        _codebooks_body,
        out_shape=[jax.ShapeDtypeStruct((ADJ, E), jnp.float32),
                   jax.ShapeDtypeStruct((ADJ, E), jnp.float32),
                   jax.ShapeDtypeStruct((1, ADJ), jnp.float32),
                   jax.ShapeDtypeStruct((1, ADJ), jnp.float32)],
    )(accp2, hs2, dinv, b2_2d)


def _quant_body(nrows, zf_ref, zf2_ref, ew_ref, ew2_ref, sq_ref, sq2_ref,
                zq_ref, zq2_ref, i1a_ref, i1b_ref, i2_ref, loss_ref):
    i = pl.program_id(0)
    blk = zf_ref.shape[0]
    iota = lax.broadcasted_iota(jnp.int32, (blk, ADJ), 1)
    big = jnp.int32(2 ** 30)
    nt = (((1,), (1,)), ((), ()))
    nn = (((1,), (0,)), ((), ()))

    zfb = zf_ref[...]
    mm = lax.dot_general(zfb, ew_ref[...], nt,
                         preferred_element_type=jnp.float32)
    d = jnp.sum(zfb ** 2, axis=1, keepdims=True) + sq_ref[...] - 2.0 * mm
    m1 = jnp.min(d, axis=1, keepdims=True)
    i1 = jnp.min(jnp.where(d == m1, iota, big), axis=1)
    oh1 = iota == i1[:, None]
    d2 = jnp.where(oh1, jnp.float32(jnp.inf), d)
    m2 = jnp.min(d2, axis=1, keepdims=True)
    i1b = jnp.min(jnp.where(d2 == m2, iota, big), axis=1)
    ohsum = (oh1 | (iota == i1b[:, None])).astype(jnp.float32)
    g = lax.dot_general(ohsum, ew_ref[...], nn,
                        preferred_element_type=jnp.float32)
    zq = g * 0.5
    zq_ref[...] = zfb + (zq - zfb)
    i1a_ref[...] = i1[:, None]
    i1b_ref[...] = i1b[:, None]

    zf2b = zf2_ref[...]
    mm2 = lax.dot_general(zf2b, ew2_ref[...], nt,
                          preferred_element_type=jnp.float32)
    dn = (jnp.sum(zf2b ** 2, axis=1, keepdims=True) + sq2_ref[...]
          - 2.0 * mm2)
    mn = jnp.min(dn, axis=1, keepdims=True)
    i2 = jnp.min(jnp.where(dn == mn, iota, big), axis=1)
    ohn = (iota == i2[:, None]).astype(jnp.float32)
    zq2 = lax.dot_general(ohn, ew2_ref[...], nn,
                          preferred_element_type=jnp.float32)
    zq2_ref[...] = zf2b + (zq2 - zf2b)
    i2_ref[...] = i2[:, None]

    part = jnp.sum((zq - zfb) ** 2) + jnp.sum((zq2 - zf2b) ** 2)
    contrib = part * ((1.0 + BETA) / (nrows * E))
    prev = jnp.where(i == 0, jnp.zeros((1, 1), jnp.float32), loss_ref[...])
    loss_ref[...] = prev + contrib


def _quant(zf, zf2, ew, ew2, sq, sq2):
    nrows = zf.shape[0]
    blk = 256
    grid = (nrows // blk,)
    full = lambda i: (0, 0)
    row = lambda i: (i, 0)
    return pl.pallas_call(
        functools.partial(_quant_body, nrows),
        grid=grid,
        in_specs=[pl.BlockSpec((blk, E), row),
                  pl.BlockSpec((blk, E), row),
                  pl.BlockSpec((ADJ, E), full),
                  pl.BlockSpec((ADJ, E), full),
                  pl.BlockSpec((1, ADJ), full),
                  pl.BlockSpec((1, ADJ), full)],
        out_specs=[pl.BlockSpec((blk, E), row),
                   pl.BlockSpec((blk, E), row),
                   pl.BlockSpec((blk, 1), row),
                   pl.BlockSpec((blk, 1), row),
                   pl.BlockSpec((blk, 1), row),
                   pl.BlockSpec((1, 1), full)],
        out_shape=[jax.ShapeDtypeStruct((nrows, E), jnp.float32),
                   jax.ShapeDtypeStruct((nrows, E), jnp.float32),
                   jax.ShapeDtypeStruct((nrows, 1), jnp.int32),
                   jax.ShapeDtypeStruct((nrows, 1), jnp.int32),
                   jax.ShapeDtypeStruct((nrows, 1), jnp.int32),
                   jax.ShapeDtypeStruct((1, 1), jnp.float32)],
    )(zf, zf2, ew, ew2, sq, sq2)


def kernel(z, code, edge_index, W1, b1, W2, b2):
    b = z.shape[0]
    src2d = edge_index[0].reshape(-1, EDGE_COLS)
    dst2d = edge_index[1].reshape(-1, EDGE_COLS)

    degp = _sc_degree(dst2d)
    mm1 = _mm1(code, W1)
    dinv, hs1 = _prep(degp, mm1)
    accp1 = _sc_scatter(hs1, src2d, dst2d)
    hs2 = _layer2(accp1, hs1, dinv, b1.reshape(1, E), W2)
    accp2 = _sc_scatter(hs2, src2d, dst2d)
    ew, ew2, sq, sq2 = _codebooks(accp2, hs2, dinv, b2.reshape(1, E))

    zf = jnp.transpose(z[:, :E], (0, 2, 3, 1)).reshape(-1, E)
    zf2 = jnp.transpose(z[:, E:], (0, 2, 3, 1)).reshape(-1, E)
    zq, zq2, i1a, i1b, i2, lossm = _quant(zf, zf2, ew, ew2, sq, sq2)

    h, w = z.shape[2], z.shape[3]
    z_adj_q = jnp.transpose(zq.reshape(b, h, w, E), (0, 3, 1, 2))
    z_noun_q = jnp.transpose(zq2.reshape(b, h, w, E), (0, 3, 1, 2))
    z_q = jnp.concatenate([z_adj_q, z_noun_q], axis=1)
    idx1 = jnp.concatenate([i1a, i1b], axis=1).reshape(b, -1)
    idx2 = i2.reshape(b, -1)
    loss = lossm.reshape(())
    return z_q, loss, idx1, idx2
